# baseline (speedup 1.0000x reference)
import functools

import jax
import jax.numpy as jnp
from jax import lax
from jax.experimental import pallas as pl
from jax.experimental.pallas import tpu as pltpu

N = 50000
E = 800000
H = 64
BN = 2000
BE = 2000

_F32 = jnp.float32


def _embed_body(x_ref, w_ref, b_ref, o_ref):
    o_ref[...] = (
        jnp.dot(x_ref[...], w_ref[...], preferred_element_type=_F32) + b_ref[...]
    )


def _embed(xp, Wp, b2):
    return pl.pallas_call(
        _embed_body,
        grid=(N // BN,),
        in_specs=[
            pl.BlockSpec((BN, 128), lambda i: (i, 0)),
            pl.BlockSpec((128, H), lambda i: (0, 0)),
            pl.BlockSpec((1, H), lambda i: (0, 0)),
        ],
        out_specs=pl.BlockSpec((BN, H), lambda i: (i, 0)),
        out_shape=jax.ShapeDtypeStruct((N, H), _F32),
    )(xp, Wp, b2)


def _proj_body(h_ref, wd_ref, ws_ref, pd_ref, ps_ref):
    h = h_ref[...]
    pd_ref[...] = jnp.dot(h, wd_ref[...], preferred_element_type=_F32)
    ps_ref[...] = jnp.dot(h, ws_ref[...], preferred_element_type=_F32)


def _proj(h, Wd, Ws):
    return pl.pallas_call(
        _proj_body,
        grid=(N // BN,),
        in_specs=[
            pl.BlockSpec((BN, H), lambda i: (i, 0)),
            pl.BlockSpec((H, 2 * H), lambda i: (0, 0)),
            pl.BlockSpec((H, 2 * H), lambda i: (0, 0)),
        ],
        out_specs=[
            pl.BlockSpec((BN, 2 * H), lambda i: (i, 0)),
            pl.BlockSpec((BN, 2 * H), lambda i: (i, 0)),
        ],
        out_shape=[
            jax.ShapeDtypeStruct((N, 2 * H), _F32),
            jax.ShapeDtypeStruct((N, 2 * H), _F32),
        ],
    )(h, Wd, Ws)


def _msg_body(g_ref, ea_ref, we_ref, b_ref, o_ref):
    zf = g_ref[...] + jnp.dot(ea_ref[...], we_ref[...],
                              preferred_element_type=_F32) + b_ref[...]
    zg = zf[:, :H]
    zc = zf[:, H:]
    sig = 1.0 / (1.0 + jnp.exp(-zg))
    sp = jnp.maximum(zc, 0.0) + jnp.log(1.0 + jnp.exp(-jnp.abs(zc)))
    o_ref[...] = sig * sp


def _msg(G, ea, We, b2):
    return pl.pallas_call(
        _msg_body,
        grid=(E // BE,),
        in_specs=[
            pl.BlockSpec((BE, 2 * H), lambda i: (i, 0)),
            pl.BlockSpec((BE, 16), lambda i: (i, 0)),
            pl.BlockSpec((16, 2 * H), lambda i: (0, 0)),
            pl.BlockSpec((1, 2 * H), lambda i: (0, 0)),
        ],
        out_specs=pl.BlockSpec((BE, H), lambda i: (i, 0)),
        out_shape=jax.ShapeDtypeStruct((E, H), _F32),
    )(G, ea, We, b2)


def _upd_body(h_ref, a_ref, o_ref):
    t = h_ref[...] + a_ref[...]
    o_ref[...] = jnp.maximum(t, 0.0) + jnp.log(1.0 + jnp.exp(-jnp.abs(t)))


def _upd(h, agg):
    return pl.pallas_call(
        _upd_body,
        grid=(N // BN,),
        in_specs=[
            pl.BlockSpec((BN, H), lambda i: (i, 0)),
            pl.BlockSpec((BN, H), lambda i: (i, 0)),
        ],
        out_specs=pl.BlockSpec((BN, H), lambda i: (i, 0)),
        out_shape=jax.ShapeDtypeStruct((N, H), _F32),
    )(h, agg)


def _gather(Pd, Ps, dst, src):
    return Pd[dst] + Ps[src]


def _scatter(M, dst):
    return jax.ops.segment_sum(M, dst, num_segments=N)


def kernel(x, edge_index, edge_attr, W_embed, b_embed,
           W_full_0, b_full_0, W_full_1, b_full_1, W_full_2, b_full_2):
    src = edge_index[0]
    dst = edge_index[1]
    xp = jnp.pad(x, ((0, 0), (0, 128 - x.shape[1])))
    Wp = jnp.pad(W_embed, ((0, 128 - W_embed.shape[0]), (0, 0)))
    h = _embed(xp, Wp, b_embed.reshape(1, H))
    for W, b in ((W_full_0, b_full_0), (W_full_1, b_full_1), (W_full_2, b_full_2)):
        Wd, Ws, We = W[:H], W[H:2 * H], W[2 * H:]
        Pd, Ps = _proj(h, Wd, Ws)
        G = _gather(Pd, Ps, dst, src)
        M = _msg(G, edge_attr, We, b.reshape(1, 2 * H))
        agg = _scatter(M, dst)
        h = _upd(h, agg)
    return h


# speedup vs baseline: 3.2073x; 3.2073x over previous
import functools

import jax
import jax.numpy as jnp
from jax import lax
from jax.experimental import pallas as pl
from jax.experimental.pallas import tpu as pltpu
from jax.experimental.pallas import tpu_sc as plsc

N = 50000
E = 800000
H = 64
BN = 2000
BE = 1600

_F32 = jnp.float32

_NC = 2
_NS = 16
_NW = _NC * _NS
_GCH = 128

_NHALF = N // _NC
_PROWS = _NHALF // 2
_DUMMY = _PROWS
_RSH = 12544
_ZR = _RSH // _NS
_SCH = 96
_OROWS = 784

_EA = 409600
_EB = E - _EA

_sc_mesh = plsc.VectorSubcoreMesh(core_axis_name="c", subcore_axis_name="s")


def _embed_body(x_ref, w_ref, b_ref, o_ref):
    o_ref[...] = (
        jnp.dot(x_ref[...], w_ref[...], preferred_element_type=_F32) + b_ref[...]
    )


def _embed(xp, Wp, b2):
    return pl.pallas_call(
        _embed_body,
        grid=(N // BN,),
        in_specs=[
            pl.BlockSpec((BN, 128), lambda i: (i, 0)),
            pl.BlockSpec((128, H), lambda i: (0, 0)),
            pl.BlockSpec((1, H), lambda i: (0, 0)),
        ],
        out_specs=pl.BlockSpec((BN, H), lambda i: (i, 0)),
        out_shape=jax.ShapeDtypeStruct((N, H), _F32),
    )(xp, Wp, b2)


def _proj_body(h_ref, wd_ref, ws_ref, pd_ref, ps_ref):
    h = h_ref[...]
    pd_ref[...] = jnp.dot(h, wd_ref[...], preferred_element_type=_F32)
    ps_ref[...] = jnp.dot(h, ws_ref[...], preferred_element_type=_F32)


def _proj(h, Wd, Ws):
    return pl.pallas_call(
        _proj_body,
        grid=(N // BN,),
        in_specs=[
            pl.BlockSpec((BN, H), lambda i: (i, 0)),
            pl.BlockSpec((H, 2 * H), lambda i: (0, 0)),
            pl.BlockSpec((H, 2 * H), lambda i: (0, 0)),
        ],
        out_specs=[
            pl.BlockSpec((BN, 2 * H), lambda i: (i, 0)),
            pl.BlockSpec((BN, 2 * H), lambda i: (i, 0)),
        ],
        out_shape=[
            jax.ShapeDtypeStruct((N, 2 * H), _F32),
            jax.ShapeDtypeStruct((N, 2 * H), _F32),
        ],
    )(h, Wd, Ws)


def _msg_body(g_ref, ea_ref, we_ref, b_ref, par_ref, o_ref):
    zf = g_ref[...] + jnp.dot(ea_ref[...], we_ref[...],
                              preferred_element_type=_F32) + b_ref[...]
    zg = zf[:, :H]
    zc = zf[:, H:]
    sig = 1.0 / (1.0 + jnp.exp(-zg))
    sp = jnp.maximum(zc, 0.0) + jnp.log(1.0 + jnp.exp(-jnp.abs(zc)))
    m = sig * sp
    par = par_ref[...]
    o_ref[...] = jnp.concatenate([m * (1.0 - par), m * par], axis=1)


def _make_msg(ne):
    def call(G, ea, We, b2, par):
        return pl.pallas_call(
            _msg_body,
            grid=(ne // BE,),
            in_specs=[
                pl.BlockSpec((BE, 2 * H), lambda i: (i, 0)),
                pl.BlockSpec((BE, 16), lambda i: (i, 0)),
                pl.BlockSpec((16, 2 * H), lambda i: (0, 0)),
                pl.BlockSpec((1, 2 * H), lambda i: (0, 0)),
                pl.BlockSpec((BE, 1), lambda i: (i, 0)),
            ],
            out_specs=pl.BlockSpec((BE, 2 * H), lambda i: (i, 0)),
            out_shape=jax.ShapeDtypeStruct((ne, 2 * H), _F32),
        )(G, ea, We, b2, par)
    return call


def _upd_body(h_ref, a_ref, b_ref, o_ref):
    t = h_ref[...] + a_ref[...] + b_ref[...]
    o_ref[...] = jnp.maximum(t, 0.0) + jnp.log(1.0 + jnp.exp(-jnp.abs(t)))


def _upd(h, aggA, aggB):
    return pl.pallas_call(
        _upd_body,
        grid=(N // BN,),
        in_specs=[
            pl.BlockSpec((BN, H), lambda i: (i, 0)),
            pl.BlockSpec((BN, H), lambda i: (i, 0)),
            pl.BlockSpec((BN, H), lambda i: (i, 0)),
        ],
        out_specs=pl.BlockSpec((BN, H), lambda i: (i, 0)),
        out_shape=jax.ShapeDtypeStruct((N, H), _F32),
    )(h, aggA, aggB)


def _make_gather(ne):
    epw = ne // _NW
    nfull = epw // _GCH
    grem = epw - nfull * _GCH
    npair = (nfull - 1) // 2
    k0 = 2 * npair

    @functools.partial(
        pl.kernel,
        out_type=jax.ShapeDtypeStruct((ne, 2 * H), _F32),
        mesh=_sc_mesh,
        scratch_types=[
            pltpu.VMEM((epw,), jnp.int32),
            pltpu.VMEM((epw,), jnp.int32),
            pltpu.VMEM((_GCH, 2 * H), _F32),
            pltpu.VMEM((_GCH, 2 * H), _F32),
            pltpu.SemaphoreType.DMA,
            pltpu.SemaphoreType.DMA,
            pltpu.SemaphoreType.DMA,
        ],
    )
    def gather_k(pd_hbm, ps_hbm, dst_hbm, src_hbm, out_hbm,
                 dsti, srci, buf0, buf1, semA0, semA1, semB):
        wid = lax.axis_index("s") * _NC + lax.axis_index("c")
        base = wid * epw
        pltpu.sync_copy(dst_hbm.at[pl.ds(base, epw)], dsti)
        pltpu.sync_copy(src_hbm.at[pl.ds(base, epw)], srci)

        bufs = (buf0, buf1)
        semsA = (semA0, semA1)

        def issue_a(off, slot):
            pltpu.async_copy(pd_hbm.at[dsti.at[pl.ds(off, _GCH)]],
                             bufs[slot], semsA[slot])

        def wait_a(off, slot):
            pltpu.make_async_copy(pd_hbm.at[dsti.at[pl.ds(off, _GCH)]],
                                  bufs[slot], semsA[slot]).wait()

        def finish(off, slot):
            pltpu.async_copy(ps_hbm.at[srci.at[pl.ds(off, _GCH)]],
                             bufs[slot], semB, add=True).wait()
            pltpu.sync_copy(bufs[slot], out_hbm.at[pl.ds(base + off, _GCH)])

        issue_a(0, 0)

        @pl.loop(0, npair)
        def _(j):
            a = (2 * j) * _GCH
            b = (2 * j + 1) * _GCH
            issue_a(b, 1)
            wait_a(a, 0)
            finish(a, 0)
            issue_a(b + _GCH, 0)
            wait_a(b, 1)
            finish(b, 1)

        wait_a(k0 * _GCH, 0)
        finish(k0 * _GCH, 0)
        if k0 + 1 < nfull:
            issue_a((k0 + 1) * _GCH, 1)
            wait_a((k0 + 1) * _GCH, 1)
            finish((k0 + 1) * _GCH, 1)
        if grem:
            roff = nfull * _GCH
            pltpu.async_copy(pd_hbm.at[dsti.at[pl.ds(roff, grem)]],
                             buf0.at[pl.ds(0, grem)], semA0).wait()
            pltpu.async_copy(ps_hbm.at[srci.at[pl.ds(roff, grem)]],
                             buf0.at[pl.ds(0, grem)], semB, add=True).wait()
            pltpu.sync_copy(buf0.at[pl.ds(0, grem)],
                            out_hbm.at[pl.ds(base + roff, grem)])

    return gather_k


def _make_scatter(ne):
    etp = ne // _NS
    sfull = etp // _SCH
    srem = etp - sfull * _SCH
    npair = (sfull - 1) // 2
    k0 = 2 * npair

    @functools.partial(
        pl.kernel,
        out_type=jax.ShapeDtypeStruct((2 * _RSH, 2 * H), _F32),
        mesh=_sc_mesh,
        scratch_types=[
            pltpu.VMEM((_SCH,), jnp.int32),
            pltpu.VMEM((_SCH,), jnp.int32),
            pltpu.VMEM((_SCH, 2 * H), _F32),
            pltpu.VMEM((_SCH, 2 * H), _F32),
            pltpu.VMEM((_SCH,), jnp.int32),
            pltpu.VMEM_SHARED((_RSH, 2 * H), _F32),
            pltpu.SemaphoreType.DMA,
            pltpu.SemaphoreType.DMA,
            pltpu.SemaphoreType.DMA,
            pltpu.SemaphoreType.DMA,
        ],
    )
    def scatter_k(msg_hbm, dst_hbm, zeros_hbm, out_hbm,
                  idx0, idx1, mbuf0, mbuf1, sidx, aggsh,
                  semM0, semM1, semI0, semI1):
        c = lax.axis_index("c")
        s = lax.axis_index("s")
        nbase = c * _NHALF
        obase = c * _RSH
        ebase = s * etp
        pltpu.sync_copy(zeros_hbm, mbuf0)
        for z in range(_ZR // _SCH):
            pltpu.sync_copy(mbuf0, aggsh.at[pl.ds(s * _ZR + z * _SCH, _SCH)])
        pltpu.sync_copy(mbuf0.at[pl.ds(0, _ZR - (_ZR // _SCH) * _SCH)],
                        aggsh.at[pl.ds(s * _ZR + (_ZR // _SCH) * _SCH,
                                       _ZR - (_ZR // _SCH) * _SCH)])
        plsc.subcore_barrier()

        mbufs = (mbuf0, mbuf1)
        idxs = (idx0, idx1)
        semsM = (semM0, semM1)
        semsI = (semI0, semI1)

        def issue_l(k, slot):
            goff = ebase + k * _SCH
            pltpu.async_copy(msg_hbm.at[pl.ds(goff, _SCH)], mbufs[slot], semsM[slot])
            pltpu.async_copy(dst_hbm.at[pl.ds(goff, _SCH)], idxs[slot], semsI[slot])

        def wait_l(k, slot):
            goff = ebase + k * _SCH
            pltpu.make_async_copy(msg_hbm.at[pl.ds(goff, _SCH)],
                                  mbufs[slot], semsM[slot]).wait()
            pltpu.make_async_copy(dst_hbm.at[pl.ds(goff, _SCH)],
                                  idxs[slot], semsI[slot]).wait()

        def finish(slot, nvec):
            for v in range(nvec):
                d = idxs[slot][pl.ds(v * 16, 16)]
                li = d - nbase
                ok = (li >= 0) & (li < _NHALF)
                sidx[pl.ds(v * 16, 16)] = jnp.where(ok, li >> 1, _DUMMY)
            for v in range(nvec, _SCH // 16):
                sidx[pl.ds(v * 16, 16)] = jnp.full((16,), _DUMMY, jnp.int32)
            pltpu.sync_copy(mbufs[slot], aggsh.at[sidx], add=True)

        issue_l(0, 0)

        @pl.loop(0, npair)
        def _(j):
            a = 2 * j
            issue_l(a + 1, 1)
            wait_l(a, 0)
            finish(0, _SCH // 16)
            issue_l(a + 2, 0)
            wait_l(a + 1, 1)
            finish(1, _SCH // 16)

        wait_l(k0, 0)
        finish(0, _SCH // 16)
        if k0 + 1 < sfull:
            issue_l(k0 + 1, 1)
            wait_l(k0 + 1, 1)
            finish(1, _SCH // 16)
        if srem:
            roff = ebase + sfull * _SCH
            pltpu.async_copy(msg_hbm.at[pl.ds(roff, srem)],
                             mbuf0.at[pl.ds(0, srem)], semM0).wait()
            pltpu.async_copy(dst_hbm.at[pl.ds(roff, srem)],
                             idx0.at[pl.ds(0, srem)], semI0).wait()
            for v in range(srem // 16):
                d = idx0[pl.ds(v * 16, 16)]
                li = d - nbase
                ok = (li >= 0) & (li < _NHALF)
                sidx[pl.ds(v * 16, 16)] = jnp.where(ok, li >> 1, _DUMMY)
            for v in range(srem // 16, _SCH // 16):
                sidx[pl.ds(v * 16, 16)] = jnp.full((16,), _DUMMY, jnp.int32)
            pltpu.sync_copy(mbuf0, aggsh.at[sidx], add=True)
        plsc.subcore_barrier()

        def out_rows(roff2, rsz):
            pltpu.sync_copy(aggsh.at[pl.ds(roff2, rsz)], mbuf0.at[pl.ds(0, rsz)])
            pltpu.sync_copy(mbuf0.at[pl.ds(0, rsz)],
                            out_hbm.at[pl.ds(obase + roff2, rsz)])

        @pl.when(s < _NS - 1)
        def _():
            for z in range(_OROWS // _SCH):
                out_rows(s * _OROWS + z * _SCH, _SCH)
            out_rows(s * _OROWS + (_OROWS // _SCH) * _SCH,
                     _OROWS - (_OROWS // _SCH) * _SCH)

        @pl.when(s == _NS - 1)
        def _():
            last = 744
            for z in range(last // _SCH):
                out_rows((_NS - 1) * _OROWS + z * _SCH, _SCH)
            out_rows((_NS - 1) * _OROWS + (last // _SCH) * _SCH,
                     last - (last // _SCH) * _SCH)

    return scatter_k


_gather_a = _make_gather(_EA)
_gather_b = _make_gather(_EB)
_scatter_a = _make_scatter(_EA)
_scatter_b = _make_scatter(_EB)
_msg_a = _make_msg(_EA)
_msg_b = _make_msg(_EB)


def kernel(x, edge_index, edge_attr, W_embed, b_embed,
           W_full_0, b_full_0, W_full_1, b_full_1, W_full_2, b_full_2):
    src = edge_index[0]
    dst = edge_index[1]
    xp = jnp.pad(x, ((0, 0), (0, 128 - x.shape[1])))
    Wp = jnp.pad(W_embed, ((0, 128 - W_embed.shape[0]), (0, 0)))
    h = _embed(xp, Wp, b_embed.reshape(1, H))
    zeros_sh = jnp.zeros((_SCH, 2 * H), _F32)
    par = (dst & 1).astype(_F32).reshape(E, 1)
    dst_a, dst_b = dst[:_EA], dst[_EA:]
    src_a, src_b = src[:_EA], src[_EA:]
    ea_a, ea_b = edge_attr[:_EA], edge_attr[_EA:]
    par_a, par_b = par[:_EA], par[_EA:]
    for W, b in ((W_full_0, b_full_0), (W_full_1, b_full_1), (W_full_2, b_full_2)):
        Wd, Ws, We = W[:H], W[H:2 * H], W[2 * H:]
        b2 = b.reshape(1, 2 * H)
        Pd, Ps = _proj(h, Wd, Ws)
        Ga = _gather_a(Pd, Ps, dst_a, src_a)
        Gb = _gather_b(Pd, Ps, dst_b, src_b)
        Ma = _msg_a(Ga, ea_a, We, b2, par_a)
        Mb = _msg_b(Gb, ea_b, We, b2, par_b)
        apA = _scatter_a(Ma, dst_a, zeros_sh)
        apB = _scatter_b(Mb, dst_b, zeros_sh)
        aggA = jnp.concatenate(
            [apA[:_PROWS], apA[_RSH:_RSH + _PROWS]], axis=0).reshape(N, H)
        aggB = jnp.concatenate(
            [apB[:_PROWS], apB[_RSH:_RSH + _PROWS]], axis=0).reshape(N, H)
        h = _upd(h, aggA, aggB)
    return h
